# Initial kernel scaffold; baseline (speedup 1.0000x reference)
import jax
import jax.numpy as jnp
from jax.experimental import pallas as pl


def kernel(x, edge_index, edge_mask, vertex_cnt, rule_cnt, W1, b1, W2, b2):
    raise NotImplementedError("write your pallas kernel here")


# speedup vs baseline: 1.0075x; 1.0075x over previous
import functools

import jax
import jax.numpy as jnp
from jax.experimental import pallas as pl
from jax.experimental.pallas import tpu as pltpu

K_STEPS = 10
ALPHA = 0.1


def _linear_body(x_ref, w1_ref, b1_ref, w2_ref, b2_ref, o_ref):
    h1 = jax.lax.dot_general(
        x_ref[...], w1_ref[...], (((1,), (1,)), ((), ())),
        preferred_element_type=jnp.float32) + b1_ref[...]
    o_ref[...] = jax.lax.dot_general(
        h1, w2_ref[...], (((1,), (1,)), ((), ())),
        preferred_element_type=jnp.float32) + b2_ref[...]


def _linear(x, W1, b1, W2, b2):
    n, d_in = x.shape
    d_out = W2.shape[0]
    blk = 1000
    grid = (n // blk,)
    return pl.pallas_call(
        _linear_body,
        grid=grid,
        in_specs=[
            pl.BlockSpec((blk, d_in), lambda i: (i, 0)),
            pl.BlockSpec((d_in, d_in), lambda i: (0, 0)),
            pl.BlockSpec((d_in,), lambda i: (0,)),
            pl.BlockSpec((d_out, d_in), lambda i: (0, 0)),
            pl.BlockSpec((d_out,), lambda i: (0,)),
        ],
        out_specs=pl.BlockSpec((blk, d_out), lambda i: (i, 0)),
        out_shape=jax.ShapeDtypeStruct((n, d_out), jnp.float32),
    )(x, W1, b1, W2, b2)


def kernel(x, edge_index, edge_mask, vertex_cnt, rule_cnt, W1, b1, W2, b2):
    x = _linear(x, W1, b1, W2, b2)
    num_segments = x.shape[0]
    h = x
    src = edge_index[0]
    dst = edge_index[1]
    m = edge_mask.astype(x.dtype)[:, None]
    for _ in range(K_STEPS):
        msgs = jnp.take(x, src, axis=0) * m
        x = jax.ops.segment_sum(msgs, dst, num_segments=num_segments)
        x = x * (1.0 - ALPHA)
        x = x + ALPHA * h
    return x


# speedup vs baseline: 2.9667x; 2.9445x over previous
import functools

import jax
import jax.numpy as jnp
from jax import lax
from jax.experimental import pallas as pl
from jax.experimental.pallas import tpu as pltpu
from jax.experimental.pallas import tpu_sc as plsc

K_STEPS = 10
ALPHA = 0.1

N_NODES = 10000
N_ACC = 10240
D = 128
CHUNK = 128
N_WORKERS = 32
ROWS_PER_SUB = N_ACC // 16
CHUNKS_PER_SUB = ROWS_PER_SUB // CHUNK


def _linear_body(x_ref, w1_ref, b1_ref, w2_ref, b2_ref, o_ref):
    h1 = lax.dot_general(
        x_ref[...], w1_ref[...], (((1,), (1,)), ((), ())),
        preferred_element_type=jnp.float32) + b1_ref[...]
    o_ref[...] = lax.dot_general(
        h1, w2_ref[...], (((1,), (1,)), ((), ())),
        preferred_element_type=jnp.float32) + b2_ref[...]


def _linear(x, W1, b1, W2, b2):
    n, d_in = x.shape
    d_out = W2.shape[0]
    blk = 1000
    return pl.pallas_call(
        _linear_body,
        grid=(n // blk,),
        in_specs=[
            pl.BlockSpec((blk, d_in), lambda i: (i, 0)),
            pl.BlockSpec((d_in, d_in), lambda i: (0, 0)),
            pl.BlockSpec((d_in,), lambda i: (0,)),
            pl.BlockSpec((d_out, d_in), lambda i: (0, 0)),
            pl.BlockSpec((d_out,), lambda i: (0,)),
        ],
        out_specs=pl.BlockSpec((blk, d_out), lambda i: (i, 0)),
        out_shape=jax.ShapeDtypeStruct((n, d_out), jnp.float32),
    )(x, W1, b1, W2, b2)


def _combine_body(p0_ref, p1_ref, h_ref, o_ref):
    o_ref[...] = ((1.0 - ALPHA) * (p0_ref[...] + p1_ref[...])
                  + ALPHA * h_ref[...])


def _combine(p0, p1, h):
    n = h.shape[0]
    blk = 2000
    return pl.pallas_call(
        _combine_body,
        grid=(n // blk,),
        in_specs=[
            pl.BlockSpec((blk, D), lambda i: (i, 0)),
            pl.BlockSpec((blk, D), lambda i: (i, 0)),
            pl.BlockSpec((blk, D), lambda i: (i, 0)),
        ],
        out_specs=pl.BlockSpec((blk, D), lambda i: (i, 0)),
        out_shape=jax.ShapeDtypeStruct((n, D), jnp.float32),
    )(p0, p1, h)


def _make_scatter_kernel(chunks_per_worker):
    mesh = plsc.VectorSubcoreMesh(core_axis_name="c", subcore_axis_name="s")

    @functools.partial(
        pl.kernel, mesh=mesh,
        out_type=jax.ShapeDtypeStruct((2, N_ACC, D), jnp.float32),
        scratch_types=[
            pltpu.VMEM((chunks_per_worker, CHUNK), jnp.int32),
            pltpu.VMEM((chunks_per_worker, CHUNK), jnp.int32),
            pltpu.VMEM((CHUNK, D), jnp.float32),
            pltpu.VMEM((CHUNK, D), jnp.float32),
            pltpu.VMEM_SHARED((N_ACC, D), jnp.float32),
            pltpu.SemaphoreType.DMA,
        ],
    )
    def scatter_round(src_hbm, dst_hbm, x_hbm, out_hbm,
                      src_v, dst_v, rows_v, stage_v, acc_sh, sem):
        c = lax.axis_index("c")
        s = lax.axis_index("s")
        w = c * 16 + s

        pltpu.sync_copy(src_hbm.at[w], src_v)
        pltpu.sync_copy(dst_hbm.at[w], dst_v)

        def zrow(r, carry):
            for cc in range(D // 16):
                stage_v[r, pl.ds(cc * 16, 16)] = jnp.zeros((16,), jnp.float32)
            return carry
        lax.fori_loop(0, CHUNK, zrow, 0)

        base = s * ROWS_PER_SUB
        for t in range(CHUNKS_PER_SUB):
            pltpu.sync_copy(stage_v, acc_sh.at[pl.ds(base + t * CHUNK, CHUNK)])
        plsc.subcore_barrier()

        def chunk_step(j, carry):
            pltpu.async_copy(x_hbm.at[src_v.at[j]], rows_v, sem).wait()
            pltpu.sync_copy(rows_v, acc_sh.at[dst_v.at[j]], add=True)
            return carry
        lax.fori_loop(0, chunks_per_worker, chunk_step, 0)
        plsc.subcore_barrier()

        for t in range(CHUNKS_PER_SUB):
            pltpu.sync_copy(acc_sh.at[pl.ds(base + t * CHUNK, CHUNK)], stage_v)
            pltpu.sync_copy(stage_v, out_hbm.at[c].at[pl.ds(base + t * CHUNK, CHUNK)])

    return scatter_round


def kernel(x, edge_index, edge_mask, vertex_cnt, rule_cnt, W1, b1, W2, b2):
    x = _linear(x, W1, b1, W2, b2)
    h = x

    src = edge_index[0].astype(jnp.int32)
    dst = edge_index[1].astype(jnp.int32)
    dst_eff = jnp.where(edge_mask, dst, N_NODES)

    n_edges = src.shape[0]
    chunks_per_worker = -(-n_edges // (N_WORKERS * CHUNK))
    e_pad = N_WORKERS * CHUNK * chunks_per_worker
    pad = e_pad - n_edges
    src_p = jnp.concatenate(
        [src, jnp.zeros((pad,), jnp.int32)]).reshape(N_WORKERS, chunks_per_worker, CHUNK)
    dst_p = jnp.concatenate(
        [dst_eff, jnp.full((pad,), N_NODES, jnp.int32)]).reshape(N_WORKERS, chunks_per_worker, CHUNK)

    scatter_round = _make_scatter_kernel(chunks_per_worker)

    for _ in range(K_STEPS):
        partials = scatter_round(src_p, dst_p, x)
        x = _combine(partials[0, :N_NODES], partials[1, :N_NODES], h)
    return x


# speedup vs baseline: 3.2094x; 1.0818x over previous
import functools

import jax
import jax.numpy as jnp
from jax import lax
from jax.experimental import pallas as pl
from jax.experimental.pallas import tpu as pltpu
from jax.experimental.pallas import tpu_sc as plsc

K_STEPS = 10
ALPHA = 0.1

N_NODES = 10000
N_ACC = 10240
D = 128
CHUNK = 128
N_WORKERS = 32
ROWS_PER_SUB = N_ACC // 16
CHUNKS_PER_SUB = ROWS_PER_SUB // CHUNK


def _linear_body(x_ref, w1_ref, b1_ref, w2_ref, b2_ref, o_ref):
    h1 = lax.dot_general(
        x_ref[...], w1_ref[...], (((1,), (1,)), ((), ())),
        preferred_element_type=jnp.float32) + b1_ref[...]
    o_ref[...] = lax.dot_general(
        h1, w2_ref[...], (((1,), (1,)), ((), ())),
        preferred_element_type=jnp.float32) + b2_ref[...]


def _linear(x, W1, b1, W2, b2):
    n, d_in = x.shape
    d_out = W2.shape[0]
    blk = 1000
    return pl.pallas_call(
        _linear_body,
        grid=(n // blk,),
        in_specs=[
            pl.BlockSpec((blk, d_in), lambda i: (i, 0)),
            pl.BlockSpec((d_in, d_in), lambda i: (0, 0)),
            pl.BlockSpec((d_in,), lambda i: (0,)),
            pl.BlockSpec((d_out, d_in), lambda i: (0, 0)),
            pl.BlockSpec((d_out,), lambda i: (0,)),
        ],
        out_specs=pl.BlockSpec((blk, d_out), lambda i: (i, 0)),
        out_shape=jax.ShapeDtypeStruct((n, d_out), jnp.float32),
    )(x, W1, b1, W2, b2)


def _combine_body(p0_ref, p1_ref, h_ref, o_ref):
    o_ref[...] = ((1.0 - ALPHA) * (p0_ref[...] + p1_ref[...])
                  + ALPHA * h_ref[...])


def _combine(p0, p1, h):
    n = h.shape[0]
    blk = 2000
    return pl.pallas_call(
        _combine_body,
        grid=(n // blk,),
        in_specs=[
            pl.BlockSpec((blk, D), lambda i: (i, 0)),
            pl.BlockSpec((blk, D), lambda i: (i, 0)),
            pl.BlockSpec((blk, D), lambda i: (i, 0)),
        ],
        out_specs=pl.BlockSpec((blk, D), lambda i: (i, 0)),
        out_shape=jax.ShapeDtypeStruct((n, D), jnp.float32),
    )(p0, p1, h)


NBUF = 2


def _make_scatter_kernel(chunks_per_worker):
    mesh = plsc.VectorSubcoreMesh(core_axis_name="c", subcore_axis_name="s")
    assert chunks_per_worker % NBUF == 0
    n_groups = chunks_per_worker // NBUF

    @functools.partial(
        pl.kernel, mesh=mesh,
        out_type=jax.ShapeDtypeStruct((2, N_ACC, D), jnp.float32),
        scratch_types=[
            pltpu.VMEM((chunks_per_worker, CHUNK), jnp.int32),
            pltpu.VMEM((chunks_per_worker, CHUNK), jnp.int32),
        ] + [pltpu.VMEM((CHUNK, D), jnp.float32)] * NBUF
        + [pltpu.VMEM_SHARED((N_ACC, D), jnp.float32)]
        + [pltpu.SemaphoreType.DMA] * (2 * NBUF),
    )
    def scatter_round(src_hbm, dst_hbm, x_hbm, out_hbm,
                      src_v, dst_v, *rest):
        rows = rest[:NBUF]
        acc_sh = rest[NBUF]
        sg = rest[NBUF + 1:NBUF + 1 + NBUF]
        ss = rest[NBUF + 1 + NBUF:]
        c = lax.axis_index("c")
        s = lax.axis_index("s")
        w = c * 16 + s

        pltpu.sync_copy(src_hbm.at[w], src_v)
        pltpu.sync_copy(dst_hbm.at[w], dst_v)

        def zrow(r, carry):
            for cc in range(D // 16):
                rows[0][r, pl.ds(cc * 16, 16)] = jnp.zeros((16,), jnp.float32)
            return carry
        lax.fori_loop(0, CHUNK, zrow, 0)
        base = s * ROWS_PER_SUB
        for t in range(CHUNKS_PER_SUB):
            pltpu.sync_copy(rows[0], acc_sh.at[pl.ds(base + t * CHUNK, CHUNK)])
        plsc.subcore_barrier()

        for b in range(NBUF):
            pltpu.async_copy(x_hbm.at[src_v.at[b]], rows[b], sg[b])

        def group(g, carry):
            for b in range(NBUF):
                j = g * NBUF + b
                pltpu.make_async_copy(x_hbm.at[src_v.at[j]], rows[b], sg[b]).wait()
                pltpu.async_copy(rows[b], acc_sh.at[dst_v.at[j]], ss[b], add=True)
            for b in range(NBUF):
                jn = (g + 1) * NBUF + b
                pltpu.make_async_copy(rows[b], acc_sh.at[dst_v.at[0]], ss[b]).wait()
                pltpu.async_copy(x_hbm.at[src_v.at[jn]], rows[b], sg[b])
            return carry
        lax.fori_loop(0, n_groups - 1, group, 0)

        for b in range(NBUF):
            j = (n_groups - 1) * NBUF + b
            pltpu.make_async_copy(x_hbm.at[src_v.at[j]], rows[b], sg[b]).wait()
            pltpu.async_copy(rows[b], acc_sh.at[dst_v.at[j]], ss[b], add=True)
        for b in range(NBUF):
            pltpu.make_async_copy(rows[b], acc_sh.at[dst_v.at[0]], ss[b]).wait()
        plsc.subcore_barrier()

        for t in range(CHUNKS_PER_SUB):
            pltpu.sync_copy(acc_sh.at[pl.ds(base + t * CHUNK, CHUNK)], rows[0])
            pltpu.sync_copy(rows[0], out_hbm.at[c].at[pl.ds(base + t * CHUNK, CHUNK)])

    return scatter_round


def kernel(x, edge_index, edge_mask, vertex_cnt, rule_cnt, W1, b1, W2, b2):
    x = _linear(x, W1, b1, W2, b2)
    h = x

    src = edge_index[0].astype(jnp.int32)
    dst = edge_index[1].astype(jnp.int32)
    dst_eff = jnp.where(edge_mask, dst, N_NODES)

    n_edges = src.shape[0]
    chunks_per_worker = -(-n_edges // (N_WORKERS * CHUNK))
    e_pad = N_WORKERS * CHUNK * chunks_per_worker
    pad = e_pad - n_edges
    src_p = jnp.concatenate(
        [src, jnp.zeros((pad,), jnp.int32)]).reshape(N_WORKERS, chunks_per_worker, CHUNK)
    dst_p = jnp.concatenate(
        [dst_eff, jnp.full((pad,), N_NODES, jnp.int32)]).reshape(N_WORKERS, chunks_per_worker, CHUNK)

    scatter_round = _make_scatter_kernel(chunks_per_worker)

    for _ in range(K_STEPS):
        partials = scatter_round(src_p, dst_p, x)
        x = _combine(partials[0, :N_NODES], partials[1, :N_NODES], h)
    return x
